# baseline (speedup 1.0000x reference)
import jax
import jax.numpy as jnp
from jax.experimental import pallas as pl
from jax.experimental.pallas import tpu as pltpu

_DIM = 768
_EPS = 1e-8
_INV_SQRT_D = 1.0 / (_DIM ** 0.5)


def _body(idx_ref, h_ref, pe_ref, w_ref, o_ref):
    x = h_ref[0] + pe_ref[0]
    ssq = jnp.sum(x * x, axis=-1, keepdims=True)
    denom = jnp.sqrt(ssq) * _INV_SQRT_D + _EPS
    o_ref[0] = (x / denom) * w_ref[...]


def kernel(hidden_state, index, pos_embed, weight):
    B, S, D = hidden_state.shape
    w2d = weight.reshape(1, D)
    idx = index.astype(jnp.int32)

    s_blk = 192
    grid_spec = pltpu.PrefetchScalarGridSpec(
        num_scalar_prefetch=1,
        grid=(B, S // s_blk),
        in_specs=[
            pl.BlockSpec((1, s_blk, D), lambda b, s, idx_ref: (b, s, 0)),
            pl.BlockSpec((1, 1, D), lambda b, s, idx_ref: (idx_ref[b], 0, 0)),
            pl.BlockSpec((1, D), lambda b, s, idx_ref: (0, 0)),
        ],
        out_specs=pl.BlockSpec((1, s_blk, D), lambda b, s, idx_ref: (b, s, 0)),
    )
    return pl.pallas_call(
        _body,
        grid_spec=grid_spec,
        out_shape=jax.ShapeDtypeStruct((B, S, D), jnp.float32),
    )(idx, hidden_state, pos_embed, w2d)
